# baseline (speedup 1.0000x reference)
import functools

import jax
import jax.numpy as jnp
from jax import lax
from jax.experimental import pallas as pl
from jax.experimental.pallas import tpu as pltpu
from jax.experimental.pallas import tpu_sc as plsc

N = 10000
E = 320000
F = 128
HEADS = 8

NC = 2
NS = 16
BATCH = 128
E_PAD = 327680
PER_S = E_PAD // NS
NBS = PER_S // BATCH
NBUF = 5
NROUND = NBS // NBUF

@functools.cache
def _sc_mesh():
    return plsc.VectorSubcoreMesh(core_axis_name="c", subcore_axis_name="s")


def _sc_gather(xl, xr, src2, dst2):
    PER_T = E_PAD // (NC * NS)
    NBT = PER_T // BATCH
    GB = 2
    NRT = NBT // GB

    @functools.partial(
        pl.kernel, mesh=_sc_mesh(),
        out_type=[jax.ShapeDtypeStruct((E_PAD, F), jnp.float32),
                  jax.ShapeDtypeStruct((E_PAD, F), jnp.float32)],
        scratch_types=[pltpu.VMEM((NBT, BATCH), jnp.int32),
                       pltpu.VMEM((NBT, BATCH), jnp.int32),
                       pltpu.VMEM((GB * BATCH, F), jnp.float32),
                       pltpu.VMEM((GB * BATCH, F), jnp.float32)]
                      + [pltpu.SemaphoreType.DMA] * (4 * GB),
    )
    def k(xl_hbm, xr_hbm, src_hbm, dst_hbm, xlo_hbm, xro_hbm,
          sidx, didx, bufa, bufb, *sems):
        c = lax.axis_index("c")
        s = lax.axis_index("s")
        wid = s * NC + c
        gsa = sems[:GB]
        gsb = sems[GB:2 * GB]
        osa = sems[2 * GB:3 * GB]
        osb = sems[3 * GB:]

        pltpu.sync_copy(src_hbm.at[pl.ds(wid * NBT, NBT)], sidx)
        pltpu.sync_copy(dst_hbm.at[pl.ds(wid * NBT, NBT)], didx)

        def slab(buf, j):
            return buf.at[pl.ds(j * BATCH, BATCH)]

        def fire(b, j):
            pltpu.async_copy(xl_hbm.at[sidx.at[b]], slab(bufa, j), gsa[j])
            pltpu.async_copy(xr_hbm.at[didx.at[b]], slab(bufb, j), gsb[j])

        for j in range(GB):
            fire(j, j)

        def round_body(t, carry):
            for j in range(GB):
                b = t * GB + j
                out = pl.ds(wid * PER_T + b * BATCH, BATCH)
                pltpu.make_async_copy(xl_hbm.at[sidx.at[b]], slab(bufa, j),
                                      gsa[j]).wait()
                pltpu.async_copy(slab(bufa, j), xlo_hbm.at[out], osa[j])
                pltpu.make_async_copy(xr_hbm.at[didx.at[b]], slab(bufb, j),
                                      gsb[j]).wait()
                pltpu.async_copy(slab(bufb, j), xro_hbm.at[out], osb[j])
            for j in range(GB):
                b = t * GB + j
                out = pl.ds(wid * PER_T + b * BATCH, BATCH)
                pltpu.make_async_copy(slab(bufa, j), xlo_hbm.at[out],
                                      osa[j]).wait()
                pltpu.make_async_copy(slab(bufb, j), xro_hbm.at[out],
                                      osb[j]).wait()

                @pl.when(b + GB < NBT)
                def _():
                    fire(b + GB, j)
            return carry

        lax.fori_loop(0, NRT, round_body, 0)

    return k(xl, xr, src2, dst2)


def _sc_scatter(ext_msg, ext_wb, dst2, zinit):
    SB = 2
    NR2 = NBS // SB

    @functools.partial(
        pl.kernel, mesh=_sc_mesh(),
        out_type=jax.ShapeDtypeStruct((2 * N, F), jnp.float32),
        scratch_types=[pltpu.VMEM((SB, BATCH), jnp.int32),
                       pltpu.VMEM((SB * BATCH, F), jnp.float32),
                       pltpu.VMEM_SHARED((N, F), jnp.float32)]
                      + [pltpu.SemaphoreType.DMA] * (3 * SB),
    )
    def k(msg_hbm, wb_hbm, dst_hbm, z_hbm, acc_hbm, idx2, rowbuf, shared,
          *sems):
        c = lax.axis_index("c")
        s = lax.axis_index("s")
        isem = sems[:SB]
        rsem = sems[SB:2 * SB]
        wsem = sems[2 * SB:]
        rows = 1000

        @pl.when(s < 10)
        def _():
            pltpu.sync_copy(z_hbm.at[pl.ds(s * rows, rows)],
                            shared.at[pl.ds(s * rows, rows)])
        plsc.subcore_barrier()

        def run(src_hbm):
            def slab(j):
                return rowbuf.at[pl.ds(j * BATCH, BATCH)]

            def fire_read(b, j):
                pltpu.async_copy(dst_hbm.at[s * NBS + b], idx2.at[j], isem[j])
                pltpu.async_copy(
                    src_hbm.at[pl.ds(s * PER_S + b * BATCH, BATCH)],
                    slab(j), rsem[j])

            for j in range(SB):
                fire_read(j, j)

            def round_body(t, carry):
                for j in range(SB):
                    b = t * SB + j
                    pltpu.make_async_copy(dst_hbm.at[s * NBS + b], idx2.at[j],
                                          isem[j]).wait()
                    pltpu.make_async_copy(
                        src_hbm.at[pl.ds(s * PER_S + b * BATCH, BATCH)],
                        slab(j), rsem[j]).wait()
                    pltpu.async_copy(slab(j), shared.at[idx2.at[j]], wsem[j],
                                     add=True)
                for j in range(SB):
                    b = t * SB + j
                    pltpu.make_async_copy(slab(j), shared.at[idx2.at[j]],
                                          wsem[j]).wait()

                    @pl.when(b + SB < NBS)
                    def _():
                        fire_read(b + SB, j)
                return carry

            lax.fori_loop(0, NR2, round_body, 0)

        @pl.when(c == 0)
        def _():
            run(msg_hbm)

        @pl.when(c == 1)
        def _():
            run(wb_hbm)

        plsc.subcore_barrier()

        @pl.when(s < 10)
        def _():
            pltpu.sync_copy(shared.at[pl.ds(s * rows, rows)],
                            acc_hbm.at[pl.ds(c * N + s * rows, rows)])

    return k(ext_msg, ext_wb, dst2, zinit)


def _mm2(x, Wl, bl, Wr, br):
    BN = 1000

    def body(x_ref, wl_ref, bl_ref, wr_ref, br_ref, xl_ref, xr_ref):
        xb = x_ref[...]
        xl_ref[...] = jnp.dot(xb, wl_ref[...],
                              preferred_element_type=jnp.float32) + bl_ref[...]
        xr_ref[...] = jnp.dot(xb, wr_ref[...],
                              preferred_element_type=jnp.float32) + br_ref[...]

    full = pl.BlockSpec((F, F), lambda i: (0, 0))
    bias = pl.BlockSpec((1, F), lambda i: (0, 0))
    return pl.pallas_call(
        body,
        grid=(N // BN,),
        in_specs=[pl.BlockSpec((BN, F), lambda i: (i, 0)), full, bias, full, bias],
        out_specs=[pl.BlockSpec((BN, F), lambda i: (i, 0)),
                   pl.BlockSpec((BN, F), lambda i: (i, 0))],
        out_shape=[jax.ShapeDtypeStruct((N, F), jnp.float32),
                   jax.ShapeDtypeStruct((N, F), jnp.float32)],
    )(x, Wl, bl, Wr, br)


def _edge(XL, XR, attrow, S, ST):
    BE = 4096

    def body(xl_ref, xr_ref, att_ref, s_ref, st_ref, msg_ref, wb_ref):
        i = pl.program_id(0)
        a = xl_ref[...]
        s = a + xr_ref[...]
        lr = jnp.where(s > 0, s, 0.2 * s)
        p = lr * att_ref[...]
        alpha = jnp.dot(p, s_ref[...], preferred_element_type=jnp.float32)
        rows = lax.broadcasted_iota(jnp.int32, (BE, 1), 0) + i * BE
        w = jnp.where(rows < E, jnp.exp(alpha), 0.0)
        wb = jnp.dot(w, st_ref[...], preferred_element_type=jnp.float32)
        msg_ref[...] = a * wb
        wb_ref[...] = wb

    return pl.pallas_call(
        body,
        grid=(E_PAD // BE,),
        in_specs=[pl.BlockSpec((BE, F), lambda i: (i, 0)),
                  pl.BlockSpec((BE, F), lambda i: (i, 0)),
                  pl.BlockSpec((1, F), lambda i: (0, 0)),
                  pl.BlockSpec((F, HEADS), lambda i: (0, 0)),
                  pl.BlockSpec((HEADS, F), lambda i: (0, 0))],
        out_specs=[pl.BlockSpec((BE, F), lambda i: (i, 0)),
                   pl.BlockSpec((BE, F), lambda i: (i, 0))],
        out_shape=[jax.ShapeDtypeStruct((E_PAD, F), jnp.float32),
                   jax.ShapeDtypeStruct((E_PAD, F), jnp.float32)],
    )(XL, XR, attrow, S, ST)


def _comb_mm(num, den, Wl, bl, Wr, br):
    BN = 1000

    def body(n_ref, d_ref, wl_ref, bl_ref, wr_ref, br_ref, xl_ref, xr_ref):
        h = n_ref[...] / (d_ref[...] + 1e-16)
        xl_ref[...] = jnp.dot(h, wl_ref[...],
                              preferred_element_type=jnp.float32) + bl_ref[...]
        xr_ref[...] = jnp.dot(h, wr_ref[...],
                              preferred_element_type=jnp.float32) + br_ref[...]

    full = pl.BlockSpec((F, F), lambda i: (0, 0))
    bias = pl.BlockSpec((1, F), lambda i: (0, 0))
    return pl.pallas_call(
        body,
        grid=(N // BN,),
        in_specs=[pl.BlockSpec((BN, F), lambda i: (i, 0)),
                  pl.BlockSpec((BN, F), lambda i: (i, 0)),
                  full, bias, full, bias],
        out_specs=[pl.BlockSpec((BN, F), lambda i: (i, 0)),
                   pl.BlockSpec((BN, F), lambda i: (i, 0))],
        out_shape=[jax.ShapeDtypeStruct((N, F), jnp.float32),
                   jax.ShapeDtypeStruct((N, F), jnp.float32)],
    )(num, den, Wl, bl, Wr, br)


def _finalize(num, den, bias):
    BN = 1000

    def body(n_ref, d_ref, b_ref, z_ref):
        h = n_ref[...] / (d_ref[...] + 1e-16) + b_ref[...]
        nrm = jnp.sqrt(jnp.sum(h * h, axis=1, keepdims=True))
        z_ref[...] = h / jnp.maximum(nrm, 1e-12)

    return pl.pallas_call(
        body,
        grid=(N // BN,),
        in_specs=[pl.BlockSpec((BN, F), lambda i: (i, 0)),
                  pl.BlockSpec((BN, F), lambda i: (i, 0)),
                  pl.BlockSpec((1, F), lambda i: (0, 0))],
        out_specs=pl.BlockSpec((BN, F), lambda i: (i, 0)),
        out_shape=jax.ShapeDtypeStruct((N, F), jnp.float32),
    )(num, den, bias)


def _decode(z):
    BR = 200

    def body(zx_ref, zy_ref, out_ref):
        zz = lax.dot_general(zx_ref[...], zy_ref[...],
                             (((1,), (1,)), ((), ())),
                             preferred_element_type=jnp.float32)
        out_ref[...] = jax.nn.sigmoid(zz)

    return pl.pallas_call(
        body,
        grid=(N // BR,),
        in_specs=[pl.BlockSpec((BR, F), lambda i: (i, 0)),
                  pl.BlockSpec((N, F), lambda i: (0, 0))],
        out_specs=pl.BlockSpec((BR, N), lambda i: (i, 0)),
        out_shape=jax.ShapeDtypeStruct((N, N), jnp.float32),
    )(z, z)


def kernel(x, edge_index, Wl1, bl1, Wr1, br1, att1, bias1,
           Wl2, bl2, Wr2, br2, att2, bias2):
    src = edge_index[0]
    dst = edge_index[1]
    pad = jnp.zeros((E_PAD - E,), jnp.int32)
    src2 = jnp.concatenate([src, pad]).reshape(E_PAD // BATCH, BATCH)
    dst2 = jnp.concatenate([dst, pad]).reshape(E_PAD // BATCH, BATCH)

    S1 = jnp.kron(jnp.eye(HEADS, dtype=jnp.float32),
                  jnp.ones((F // HEADS, 1), jnp.float32))
    ST1 = S1.T
    S2 = jnp.ones((F, HEADS), jnp.float32)
    ST2 = jnp.ones((HEADS, F), jnp.float32) / HEADS
    zinit = jnp.zeros((N, F), jnp.float32)

    xl1, xr1 = _mm2(x, Wl1, bl1.reshape(1, F), Wr1, br1.reshape(1, F))
    XL1, XR1 = _sc_gather(xl1, xr1, src2, dst2)
    msg1, wb1 = _edge(XL1, XR1, att1.reshape(1, F), S1, ST1)
    acc1 = _sc_scatter(msg1, wb1, dst2, zinit)
    xl2, xr2 = _comb_mm(acc1[:N], acc1[N:], Wl2, bl2.reshape(1, F),
                        Wr2, br2.reshape(1, F))
    XL2, XR2 = _sc_gather(xl2, xr2, src2, dst2)
    msg2, wb2 = _edge(XL2, XR2, att2.reshape(1, F), S2, ST2)
    acc2 = _sc_scatter(msg2, wb2, dst2, zinit)
    z = _finalize(acc2[:N], acc2[N:], bias2.reshape(1, F))
    A = _decode(z)
    return (A, z)


# speedup vs baseline: 1.5797x; 1.5797x over previous
import functools

import jax
import jax.numpy as jnp
from jax import lax
from jax.experimental import pallas as pl
from jax.experimental.pallas import tpu as pltpu
from jax.experimental.pallas import tpu_sc as plsc

N = 10000
E = 320000
F = 128
HEADS = 8

NC = 2
NS = 16
BATCH = 128
E_PAD = 327680
PER_S = E_PAD // NS
NBS = PER_S // BATCH
NBUF = 5
NROUND = NBS // NBUF

@functools.cache
def _sc_mesh():
    return plsc.VectorSubcoreMesh(core_axis_name="c", subcore_axis_name="s")


def _sc_gather(xl, xr, src2, dst2):
    HALF = NBS // 2

    @functools.partial(
        pl.kernel, mesh=_sc_mesh(),
        out_type=[jax.ShapeDtypeStruct((E_PAD, F), jnp.float32),
                  jax.ShapeDtypeStruct((E_PAD, F), jnp.float32)],
        scratch_types=[pltpu.VMEM((HALF, BATCH), jnp.int32),
                       pltpu.VMEM((2 * BATCH, F), jnp.float32),
                       pltpu.VMEM_SHARED((N, F), jnp.float32)]
                      + [pltpu.SemaphoreType.DMA] * 4,
    )
    def k(xl_hbm, xr_hbm, src_hbm, dst_hbm, xlo_hbm, xro_hbm,
          idx2, rowbuf, table, *sems):
        c = lax.axis_index("c")
        s = lax.axis_index("s")
        gsem = sems[:2]
        osem = sems[2:]
        rows = 1000

        def run(tab_hbm, eidx_hbm, out_hbm):
            @pl.when(s < 10)
            def _():
                pltpu.sync_copy(tab_hbm.at[pl.ds(s * rows, rows)],
                                table.at[pl.ds(s * rows, rows)])
            plsc.subcore_barrier()

            def slab(j):
                return rowbuf.at[pl.ds(j * BATCH, BATCH)]

            def phase(ph):
                base = ph * HALF
                pltpu.sync_copy(
                    eidx_hbm.at[pl.ds(s * NBS + base, HALF)], idx2)

                def fire(b, j):
                    pltpu.async_copy(table.at[idx2.at[b]], slab(j), gsem[j])

                for j in range(2):
                    fire(j, j)

                def round_body(t, carry):
                    for j in range(2):
                        b = t * 2 + j
                        out = pl.ds(s * PER_S + (base + b) * BATCH, BATCH)
                        pltpu.make_async_copy(table.at[idx2.at[b]], slab(j),
                                              gsem[j]).wait()
                        pltpu.async_copy(slab(j), out_hbm.at[out], osem[j])
                    for j in range(2):
                        b = t * 2 + j
                        out = pl.ds(s * PER_S + (base + b) * BATCH, BATCH)
                        pltpu.make_async_copy(slab(j), out_hbm.at[out],
                                              osem[j]).wait()

                        @pl.when(b + 2 < HALF)
                        def _():
                            fire(b + 2, j)
                    return carry

                lax.fori_loop(0, HALF // 2, round_body, 0)

            phase(0)
            phase(1)

        @pl.when(c == 0)
        def _():
            run(xl_hbm, src_hbm, xlo_hbm)

        @pl.when(c == 1)
        def _():
            run(xr_hbm, dst_hbm, xro_hbm)

    return k(xl, xr, src2, dst2)


def _sc_scatter(ext_msg, ext_wb, dst2, zinit):
    SB = 2
    NR2 = NBS // SB

    @functools.partial(
        pl.kernel, mesh=_sc_mesh(),
        out_type=jax.ShapeDtypeStruct((2 * N, F), jnp.float32),
        scratch_types=[pltpu.VMEM((SB, BATCH), jnp.int32),
                       pltpu.VMEM((SB * BATCH, F), jnp.float32),
                       pltpu.VMEM_SHARED((N, F), jnp.float32)]
                      + [pltpu.SemaphoreType.DMA] * (3 * SB),
    )
    def k(msg_hbm, wb_hbm, dst_hbm, z_hbm, acc_hbm, idx2, rowbuf, shared,
          *sems):
        c = lax.axis_index("c")
        s = lax.axis_index("s")
        isem = sems[:SB]
        rsem = sems[SB:2 * SB]
        wsem = sems[2 * SB:]
        rows = 1000

        @pl.when(s < 10)
        def _():
            pltpu.sync_copy(z_hbm.at[pl.ds(s * rows, rows)],
                            shared.at[pl.ds(s * rows, rows)])
        plsc.subcore_barrier()

        def run(src_hbm):
            def slab(j):
                return rowbuf.at[pl.ds(j * BATCH, BATCH)]

            def fire_read(b, j):
                pltpu.async_copy(dst_hbm.at[s * NBS + b], idx2.at[j], isem[j])
                pltpu.async_copy(
                    src_hbm.at[pl.ds(s * PER_S + b * BATCH, BATCH)],
                    slab(j), rsem[j])

            for j in range(SB):
                fire_read(j, j)

            def round_body(t, carry):
                for j in range(SB):
                    b = t * SB + j
                    pltpu.make_async_copy(dst_hbm.at[s * NBS + b], idx2.at[j],
                                          isem[j]).wait()
                    pltpu.make_async_copy(
                        src_hbm.at[pl.ds(s * PER_S + b * BATCH, BATCH)],
                        slab(j), rsem[j]).wait()
                    pltpu.async_copy(slab(j), shared.at[idx2.at[j]], wsem[j],
                                     add=True)
                for j in range(SB):
                    b = t * SB + j
                    pltpu.make_async_copy(slab(j), shared.at[idx2.at[j]],
                                          wsem[j]).wait()

                    @pl.when(b + SB < NBS)
                    def _():
                        fire_read(b + SB, j)
                return carry

            lax.fori_loop(0, NR2, round_body, 0)

        @pl.when(c == 0)
        def _():
            run(msg_hbm)

        @pl.when(c == 1)
        def _():
            run(wb_hbm)

        plsc.subcore_barrier()

        @pl.when(s < 10)
        def _():
            pltpu.sync_copy(shared.at[pl.ds(s * rows, rows)],
                            acc_hbm.at[pl.ds(c * N + s * rows, rows)])

    return k(ext_msg, ext_wb, dst2, zinit)


def _mm2(x, Wl, bl, Wr, br):
    BN = 1000

    def body(x_ref, wl_ref, bl_ref, wr_ref, br_ref, xl_ref, xr_ref):
        xb = x_ref[...]
        xl_ref[...] = jnp.dot(xb, wl_ref[...],
                              preferred_element_type=jnp.float32) + bl_ref[...]
        xr_ref[...] = jnp.dot(xb, wr_ref[...],
                              preferred_element_type=jnp.float32) + br_ref[...]

    full = pl.BlockSpec((F, F), lambda i: (0, 0))
    bias = pl.BlockSpec((1, F), lambda i: (0, 0))
    return pl.pallas_call(
        body,
        grid=(N // BN,),
        in_specs=[pl.BlockSpec((BN, F), lambda i: (i, 0)), full, bias, full, bias],
        out_specs=[pl.BlockSpec((BN, F), lambda i: (i, 0)),
                   pl.BlockSpec((BN, F), lambda i: (i, 0))],
        out_shape=[jax.ShapeDtypeStruct((N, F), jnp.float32),
                   jax.ShapeDtypeStruct((N, F), jnp.float32)],
    )(x, Wl, bl, Wr, br)


def _edge(XL, XR, attrow, S, ST):
    BE = 4096

    def body(xl_ref, xr_ref, att_ref, s_ref, st_ref, msg_ref, wb_ref):
        i = pl.program_id(0)
        a = xl_ref[...]
        s = a + xr_ref[...]
        lr = jnp.where(s > 0, s, 0.2 * s)
        p = lr * att_ref[...]
        alpha = jnp.dot(p, s_ref[...], preferred_element_type=jnp.float32)
        rows = lax.broadcasted_iota(jnp.int32, (BE, 1), 0) + i * BE
        w = jnp.where(rows < E, jnp.exp(alpha), 0.0)
        wb = jnp.dot(w, st_ref[...], preferred_element_type=jnp.float32)
        msg_ref[...] = a * wb
        wb_ref[...] = wb

    return pl.pallas_call(
        body,
        grid=(E_PAD // BE,),
        in_specs=[pl.BlockSpec((BE, F), lambda i: (i, 0)),
                  pl.BlockSpec((BE, F), lambda i: (i, 0)),
                  pl.BlockSpec((1, F), lambda i: (0, 0)),
                  pl.BlockSpec((F, HEADS), lambda i: (0, 0)),
                  pl.BlockSpec((HEADS, F), lambda i: (0, 0))],
        out_specs=[pl.BlockSpec((BE, F), lambda i: (i, 0)),
                   pl.BlockSpec((BE, F), lambda i: (i, 0))],
        out_shape=[jax.ShapeDtypeStruct((E_PAD, F), jnp.float32),
                   jax.ShapeDtypeStruct((E_PAD, F), jnp.float32)],
    )(XL, XR, attrow, S, ST)


def _comb_mm(num, den, Wl, bl, Wr, br):
    BN = 1000

    def body(n_ref, d_ref, wl_ref, bl_ref, wr_ref, br_ref, xl_ref, xr_ref):
        h = n_ref[...] / (d_ref[...] + 1e-16)
        xl_ref[...] = jnp.dot(h, wl_ref[...],
                              preferred_element_type=jnp.float32) + bl_ref[...]
        xr_ref[...] = jnp.dot(h, wr_ref[...],
                              preferred_element_type=jnp.float32) + br_ref[...]

    full = pl.BlockSpec((F, F), lambda i: (0, 0))
    bias = pl.BlockSpec((1, F), lambda i: (0, 0))
    return pl.pallas_call(
        body,
        grid=(N // BN,),
        in_specs=[pl.BlockSpec((BN, F), lambda i: (i, 0)),
                  pl.BlockSpec((BN, F), lambda i: (i, 0)),
                  full, bias, full, bias],
        out_specs=[pl.BlockSpec((BN, F), lambda i: (i, 0)),
                   pl.BlockSpec((BN, F), lambda i: (i, 0))],
        out_shape=[jax.ShapeDtypeStruct((N, F), jnp.float32),
                   jax.ShapeDtypeStruct((N, F), jnp.float32)],
    )(num, den, Wl, bl, Wr, br)


def _finalize(num, den, bias):
    BN = 1000

    def body(n_ref, d_ref, b_ref, z_ref):
        h = n_ref[...] / (d_ref[...] + 1e-16) + b_ref[...]
        nrm = jnp.sqrt(jnp.sum(h * h, axis=1, keepdims=True))
        z_ref[...] = h / jnp.maximum(nrm, 1e-12)

    return pl.pallas_call(
        body,
        grid=(N // BN,),
        in_specs=[pl.BlockSpec((BN, F), lambda i: (i, 0)),
                  pl.BlockSpec((BN, F), lambda i: (i, 0)),
                  pl.BlockSpec((1, F), lambda i: (0, 0))],
        out_specs=pl.BlockSpec((BN, F), lambda i: (i, 0)),
        out_shape=jax.ShapeDtypeStruct((N, F), jnp.float32),
    )(num, den, bias)


def _decode(z):
    BR = 200

    def body(zx_ref, zy_ref, out_ref):
        zz = lax.dot_general(zx_ref[...], zy_ref[...],
                             (((1,), (1,)), ((), ())),
                             preferred_element_type=jnp.float32)
        out_ref[...] = jax.nn.sigmoid(zz)

    return pl.pallas_call(
        body,
        grid=(N // BR,),
        in_specs=[pl.BlockSpec((BR, F), lambda i: (i, 0)),
                  pl.BlockSpec((N, F), lambda i: (0, 0))],
        out_specs=pl.BlockSpec((BR, N), lambda i: (i, 0)),
        out_shape=jax.ShapeDtypeStruct((N, N), jnp.float32),
    )(z, z)


def kernel(x, edge_index, Wl1, bl1, Wr1, br1, att1, bias1,
           Wl2, bl2, Wr2, br2, att2, bias2):
    src = edge_index[0]
    dst = edge_index[1]
    pad = jnp.zeros((E_PAD - E,), jnp.int32)
    src2 = jnp.concatenate([src, pad]).reshape(E_PAD // BATCH, BATCH)
    dst2 = jnp.concatenate([dst, pad]).reshape(E_PAD // BATCH, BATCH)

    S1 = jnp.kron(jnp.eye(HEADS, dtype=jnp.float32),
                  jnp.ones((F // HEADS, 1), jnp.float32))
    ST1 = S1.T
    S2 = jnp.ones((F, HEADS), jnp.float32)
    ST2 = jnp.ones((HEADS, F), jnp.float32) / HEADS
    zinit = jnp.zeros((N, F), jnp.float32)

    xl1, xr1 = _mm2(x, Wl1, bl1.reshape(1, F), Wr1, br1.reshape(1, F))
    XL1, XR1 = _sc_gather(xl1, xr1, src2, dst2)
    msg1, wb1 = _edge(XL1, XR1, att1.reshape(1, F), S1, ST1)
    acc1 = _sc_scatter(msg1, wb1, dst2, zinit)
    xl2, xr2 = _comb_mm(acc1[:N], acc1[N:], Wl2, bl2.reshape(1, F),
                        Wr2, br2.reshape(1, F))
    XL2, XR2 = _sc_gather(xl2, xr2, src2, dst2)
    msg2, wb2 = _edge(XL2, XR2, att2.reshape(1, F), S2, ST2)
    acc2 = _sc_scatter(msg2, wb2, dst2, zinit)
    z = _finalize(acc2[:N], acc2[N:], bias2.reshape(1, F))
    A = _decode(z)
    return (A, z)


# speedup vs baseline: 1.7187x; 1.0879x over previous
import functools

import jax
import jax.numpy as jnp
from jax import lax
from jax.experimental import pallas as pl
from jax.experimental.pallas import tpu as pltpu
from jax.experimental.pallas import tpu_sc as plsc

N = 10000
E = 320000
F = 128
HEADS = 8

NC = 2
NS = 16
BATCH = 128
E_PAD = 327680
PER_S = E_PAD // NS
NBS = PER_S // BATCH
NBUF = 5
NROUND = NBS // NBUF

@functools.cache
def _sc_mesh():
    return plsc.VectorSubcoreMesh(core_axis_name="c", subcore_axis_name="s")


def _sc_gather(xl, xr, src2, dst2):
    HALF = NBS // 2

    @functools.partial(
        pl.kernel, mesh=_sc_mesh(),
        out_type=[jax.ShapeDtypeStruct((E_PAD, F), jnp.float32),
                  jax.ShapeDtypeStruct((E_PAD, F), jnp.float32)],
        scratch_types=[pltpu.VMEM((HALF, BATCH), jnp.int32),
                       pltpu.VMEM((2 * BATCH, F), jnp.float32),
                       pltpu.VMEM_SHARED((N, F), jnp.float32)]
                      + [pltpu.SemaphoreType.DMA] * 4,
    )
    def k(xl_hbm, xr_hbm, src_hbm, dst_hbm, xlo_hbm, xro_hbm,
          idx2, rowbuf, table, *sems):
        c = lax.axis_index("c")
        s = lax.axis_index("s")
        gsem = sems[:2]
        osem = sems[2:]
        rows = 1000

        def run(tab_hbm, eidx_hbm, out_hbm):
            @pl.when(s < 10)
            def _():
                pltpu.sync_copy(tab_hbm.at[pl.ds(s * rows, rows)],
                                table.at[pl.ds(s * rows, rows)])
            plsc.subcore_barrier()

            def slab(j):
                return rowbuf.at[pl.ds(j * BATCH, BATCH)]

            def phase(ph):
                base = ph * HALF
                pltpu.sync_copy(
                    eidx_hbm.at[pl.ds(s * NBS + base, HALF)], idx2)

                def fire(b, j):
                    pltpu.async_copy(table.at[idx2.at[b]], slab(j), gsem[j])

                for j in range(2):
                    fire(j, j)

                def round_body(t, carry):
                    for j in range(2):
                        b = t * 2 + j
                        out = pl.ds(s * PER_S + (base + b) * BATCH, BATCH)
                        pltpu.make_async_copy(table.at[idx2.at[b]], slab(j),
                                              gsem[j]).wait()
                        pltpu.async_copy(slab(j), out_hbm.at[out], osem[j])
                    for j in range(2):
                        b = t * 2 + j
                        out = pl.ds(s * PER_S + (base + b) * BATCH, BATCH)
                        pltpu.make_async_copy(slab(j), out_hbm.at[out],
                                              osem[j]).wait()

                        @pl.when(b + 2 < HALF)
                        def _():
                            fire(b + 2, j)
                    return carry

                lax.fori_loop(0, HALF // 2, round_body, 0)

            phase(0)
            phase(1)

        @pl.when(c == 0)
        def _():
            run(xl_hbm, src_hbm, xlo_hbm)

        @pl.when(c == 1)
        def _():
            run(xr_hbm, dst_hbm, xro_hbm)

    return k(xl, xr, src2, dst2)


def _sc_scatter(ext_msg, ext_wb, dst2, zinit):
    SB = 3
    NR2 = NBS // SB

    @functools.partial(
        pl.kernel, mesh=_sc_mesh(),
        out_type=jax.ShapeDtypeStruct((2 * N, F), jnp.float32),
        scratch_types=[pltpu.VMEM((SB, BATCH), jnp.int32),
                       pltpu.VMEM((SB * BATCH, F), jnp.float32),
                       pltpu.VMEM_SHARED((N, F), jnp.float32)]
                      + [pltpu.SemaphoreType.DMA] * (3 * SB),
    )
    def k(msg_hbm, wb_hbm, dst_hbm, z_hbm, acc_hbm, idx2, rowbuf, shared,
          *sems):
        c = lax.axis_index("c")
        s = lax.axis_index("s")
        isem = sems[:SB]
        rsem = sems[SB:2 * SB]
        wsem = sems[2 * SB:]
        rows = 1000

        @pl.when(s < 10)
        def _():
            pltpu.sync_copy(z_hbm.at[pl.ds(s * rows, rows)],
                            shared.at[pl.ds(s * rows, rows)])
        plsc.subcore_barrier()

        def run(src_hbm):
            def slab(j):
                return rowbuf.at[pl.ds(j * BATCH, BATCH)]

            def fire_read(b, j):
                pltpu.async_copy(dst_hbm.at[s * NBS + b], idx2.at[j], isem[j])
                pltpu.async_copy(
                    src_hbm.at[pl.ds(s * PER_S + b * BATCH, BATCH)],
                    slab(j), rsem[j])

            for j in range(SB):
                fire_read(j, j)

            def round_body(t, carry):
                for j in range(SB):
                    b = t * SB + j
                    pltpu.make_async_copy(dst_hbm.at[s * NBS + b], idx2.at[j],
                                          isem[j]).wait()
                    pltpu.make_async_copy(
                        src_hbm.at[pl.ds(s * PER_S + b * BATCH, BATCH)],
                        slab(j), rsem[j]).wait()
                    pltpu.async_copy(slab(j), shared.at[idx2.at[j]], wsem[j],
                                     add=True)
                for j in range(SB):
                    b = t * SB + j
                    pltpu.make_async_copy(slab(j), shared.at[idx2.at[j]],
                                          wsem[j]).wait()

                    @pl.when(b + SB < NBS)
                    def _():
                        fire_read(b + SB, j)
                return carry

            lax.fori_loop(0, NR2, round_body, 0)
            b = NR2 * SB
            pltpu.make_async_copy(dst_hbm.at[s * NBS + b], idx2.at[0],
                                  isem[0]).wait()
            pltpu.make_async_copy(
                src_hbm.at[pl.ds(s * PER_S + b * BATCH, BATCH)],
                slab(0), rsem[0]).wait()
            pltpu.sync_copy(slab(0), shared.at[idx2.at[0]], add=True)

        @pl.when(c == 0)
        def _():
            run(msg_hbm)

        @pl.when(c == 1)
        def _():
            run(wb_hbm)

        plsc.subcore_barrier()

        @pl.when(s < 10)
        def _():
            pltpu.sync_copy(shared.at[pl.ds(s * rows, rows)],
                            acc_hbm.at[pl.ds(c * N + s * rows, rows)])

    return k(ext_msg, ext_wb, dst2, zinit)


def _mm2(x, Wl, bl, Wr, br):
    BN = 1000

    def body(x_ref, wl_ref, bl_ref, wr_ref, br_ref, xl_ref, xr_ref):
        xb = x_ref[...]
        xl_ref[...] = jnp.dot(xb, wl_ref[...],
                              preferred_element_type=jnp.float32) + bl_ref[...]
        xr_ref[...] = jnp.dot(xb, wr_ref[...],
                              preferred_element_type=jnp.float32) + br_ref[...]

    full = pl.BlockSpec((F, F), lambda i: (0, 0))
    bias = pl.BlockSpec((1, F), lambda i: (0, 0))
    return pl.pallas_call(
        body,
        grid=(N // BN,),
        in_specs=[pl.BlockSpec((BN, F), lambda i: (i, 0)), full, bias, full, bias],
        out_specs=[pl.BlockSpec((BN, F), lambda i: (i, 0)),
                   pl.BlockSpec((BN, F), lambda i: (i, 0))],
        out_shape=[jax.ShapeDtypeStruct((N, F), jnp.float32),
                   jax.ShapeDtypeStruct((N, F), jnp.float32)],
    )(x, Wl, bl, Wr, br)


def _edge(XL, XR, attrow, S, ST):
    BE = 4096

    def body(xl_ref, xr_ref, att_ref, s_ref, st_ref, msg_ref, wb_ref):
        i = pl.program_id(0)
        a = xl_ref[...]
        s = a + xr_ref[...]
        lr = jnp.where(s > 0, s, 0.2 * s)
        p = lr * att_ref[...]
        alpha = jnp.dot(p, s_ref[...], preferred_element_type=jnp.float32)
        rows = lax.broadcasted_iota(jnp.int32, (BE, 1), 0) + i * BE
        w = jnp.where(rows < E, jnp.exp(alpha), 0.0)
        wb = jnp.dot(w, st_ref[...], preferred_element_type=jnp.float32)
        msg_ref[...] = a * wb
        wb_ref[...] = wb

    return pl.pallas_call(
        body,
        grid=(E_PAD // BE,),
        in_specs=[pl.BlockSpec((BE, F), lambda i: (i, 0)),
                  pl.BlockSpec((BE, F), lambda i: (i, 0)),
                  pl.BlockSpec((1, F), lambda i: (0, 0)),
                  pl.BlockSpec((F, HEADS), lambda i: (0, 0)),
                  pl.BlockSpec((HEADS, F), lambda i: (0, 0))],
        out_specs=[pl.BlockSpec((BE, F), lambda i: (i, 0)),
                   pl.BlockSpec((BE, F), lambda i: (i, 0))],
        out_shape=[jax.ShapeDtypeStruct((E_PAD, F), jnp.float32),
                   jax.ShapeDtypeStruct((E_PAD, F), jnp.float32)],
    )(XL, XR, attrow, S, ST)


def _comb_mm(num, den, Wl, bl, Wr, br):
    BN = 1000

    def body(n_ref, d_ref, wl_ref, bl_ref, wr_ref, br_ref, xl_ref, xr_ref):
        h = n_ref[...] / (d_ref[...] + 1e-16)
        xl_ref[...] = jnp.dot(h, wl_ref[...],
                              preferred_element_type=jnp.float32) + bl_ref[...]
        xr_ref[...] = jnp.dot(h, wr_ref[...],
                              preferred_element_type=jnp.float32) + br_ref[...]

    full = pl.BlockSpec((F, F), lambda i: (0, 0))
    bias = pl.BlockSpec((1, F), lambda i: (0, 0))
    return pl.pallas_call(
        body,
        grid=(N // BN,),
        in_specs=[pl.BlockSpec((BN, F), lambda i: (i, 0)),
                  pl.BlockSpec((BN, F), lambda i: (i, 0)),
                  full, bias, full, bias],
        out_specs=[pl.BlockSpec((BN, F), lambda i: (i, 0)),
                   pl.BlockSpec((BN, F), lambda i: (i, 0))],
        out_shape=[jax.ShapeDtypeStruct((N, F), jnp.float32),
                   jax.ShapeDtypeStruct((N, F), jnp.float32)],
    )(num, den, Wl, bl, Wr, br)


def _finalize(num, den, bias):
    BN = 1000

    def body(n_ref, d_ref, b_ref, z_ref):
        h = n_ref[...] / (d_ref[...] + 1e-16) + b_ref[...]
        nrm = jnp.sqrt(jnp.sum(h * h, axis=1, keepdims=True))
        z_ref[...] = h / jnp.maximum(nrm, 1e-12)

    return pl.pallas_call(
        body,
        grid=(N // BN,),
        in_specs=[pl.BlockSpec((BN, F), lambda i: (i, 0)),
                  pl.BlockSpec((BN, F), lambda i: (i, 0)),
                  pl.BlockSpec((1, F), lambda i: (0, 0))],
        out_specs=pl.BlockSpec((BN, F), lambda i: (i, 0)),
        out_shape=jax.ShapeDtypeStruct((N, F), jnp.float32),
    )(num, den, bias)


def _decode(z):
    BR = 200

    def body(zx_ref, zy_ref, out_ref):
        zz = lax.dot_general(zx_ref[...], zy_ref[...],
                             (((1,), (1,)), ((), ())),
                             preferred_element_type=jnp.float32)
        out_ref[...] = jax.nn.sigmoid(zz)

    return pl.pallas_call(
        body,
        grid=(N // BR,),
        in_specs=[pl.BlockSpec((BR, F), lambda i: (i, 0)),
                  pl.BlockSpec((N, F), lambda i: (0, 0))],
        out_specs=pl.BlockSpec((BR, N), lambda i: (i, 0)),
        out_shape=jax.ShapeDtypeStruct((N, N), jnp.float32),
    )(z, z)


def kernel(x, edge_index, Wl1, bl1, Wr1, br1, att1, bias1,
           Wl2, bl2, Wr2, br2, att2, bias2):
    src = edge_index[0]
    dst = edge_index[1]
    pad = jnp.zeros((E_PAD - E,), jnp.int32)
    src2 = jnp.concatenate([src, pad]).reshape(E_PAD // BATCH, BATCH)
    dst2 = jnp.concatenate([dst, pad]).reshape(E_PAD // BATCH, BATCH)

    S1 = jnp.kron(jnp.eye(HEADS, dtype=jnp.float32),
                  jnp.ones((F // HEADS, 1), jnp.float32))
    ST1 = S1.T
    S2 = jnp.ones((F, HEADS), jnp.float32)
    ST2 = jnp.ones((HEADS, F), jnp.float32) / HEADS
    zinit = jnp.zeros((N, F), jnp.float32)

    xl1, xr1 = _mm2(x, Wl1, bl1.reshape(1, F), Wr1, br1.reshape(1, F))
    XL1, XR1 = _sc_gather(xl1, xr1, src2, dst2)
    msg1, wb1 = _edge(XL1, XR1, att1.reshape(1, F), S1, ST1)
    acc1 = _sc_scatter(msg1, wb1, dst2, zinit)
    xl2, xr2 = _comb_mm(acc1[:N], acc1[N:], Wl2, bl2.reshape(1, F),
                        Wr2, br2.reshape(1, F))
    XL2, XR2 = _sc_gather(xl2, xr2, src2, dst2)
    msg2, wb2 = _edge(XL2, XR2, att2.reshape(1, F), S2, ST2)
    acc2 = _sc_scatter(msg2, wb2, dst2, zinit)
    z = _finalize(acc2[:N], acc2[N:], bias2.reshape(1, F))
    A = _decode(z)
    return (A, z)
